# baseline (speedup 1.0000x reference)
import functools

import jax
import jax.numpy as jnp
from jax import lax
from jax.experimental import pallas as pl
from jax.experimental.pallas import tpu as pltpu
from jax.experimental.pallas import tpu_sc as plsc

_V = 100000
_D = 64
_B = 16384
_NT = 7
_NB = 2 * _NT
_RPC = 32

_mesh = plsc.VectorSubcoreMesh(core_axis_name="c", subcore_axis_name="s")


@functools.partial(
    pl.kernel,
    mesh=_mesh,
    compiler_params=pltpu.CompilerParams(use_tc_tiling_on_sc=True),
    out_type=jax.ShapeDtypeStruct((_D, _B), jnp.float32),
    scratch_types=(
        [pltpu.VMEM_SHARED((_V,), jnp.float32) for _ in range(_NB)]
        + [
            pltpu.VMEM((_B // 2,), jnp.int32),
            pltpu.VMEM((_B // 2,), jnp.int32),
            pltpu.VMEM((_B // 2,), jnp.float32),
            pltpu.VMEM((_B // 2,), jnp.float32),
            pltpu.SemaphoreType.DMA,
            pltpu.SemaphoreType.DMA,
            pltpu.SemaphoreType.DMA,
        ]
    ),
)
def _gather_kernel(wt_hbm, idx_hbm, out_hbm, *refs):
    bufs = refs[:_NB]
    idx_a, idx_b, o_a, o_b, sem_a, sem_b, sem_s = refs[_NB:]
    c = lax.axis_index("c")
    s = lax.axis_index("s")
    pltpu.sync_copy(idx_hbm.at[pl.ds(0, _B // 2)], idx_a)
    pltpu.sync_copy(idx_hbm.at[pl.ds(_B // 2, _B // 2)], idx_b)
    for i in range(_NT):
        rows = [t for t in range(i, _RPC, _NT)]

        @pl.when(s == i)
        def _(i=i, rows=rows):
            stage = pltpu.async_copy(
                wt_hbm.at[c * _RPC + rows[0]], bufs[2 * i], sem_s
            )
            for t, r in enumerate(rows):
                d = c * _RPC + r
                cur = bufs[2 * i + (t % 2)]
                stage.wait()
                if t + 1 < len(rows):
                    stage = pltpu.async_copy(
                        wt_hbm.at[c * _RPC + rows[t + 1]],
                        bufs[2 * i + ((t + 1) % 2)],
                        sem_s,
                    )
                ga = pltpu.async_copy(cur.at[idx_a], o_a, sem_a)
                gb = pltpu.async_copy(cur.at[idx_b], o_b, sem_b)
                ga.wait()
                gb.wait()
                pltpu.sync_copy(o_a, out_hbm.at[d, pl.ds(0, _B // 2)])
                pltpu.sync_copy(o_b, out_hbm.at[d, pl.ds(_B // 2, _B // 2)])


def kernel(x, emb_weight):
    out_t = _gather_kernel(emb_weight.T, x.astype(jnp.int32))
    return out_t.T


# speedup vs baseline: 1.0223x; 1.0223x over previous
import functools

import jax
import jax.numpy as jnp
from jax import lax
from jax.experimental import pallas as pl
from jax.experimental.pallas import tpu as pltpu
from jax.experimental.pallas import tpu_sc as plsc

_V = 100000
_D = 64
_B = 16384
_NS = 16
_NB = 15
_RPC = 32
_CH = _B // _NS

_mesh = plsc.VectorSubcoreMesh(core_axis_name="c", subcore_axis_name="s")


@functools.partial(
    pl.kernel,
    mesh=_mesh,
    compiler_params=pltpu.CompilerParams(use_tc_tiling_on_sc=True),
    out_type=jax.ShapeDtypeStruct((_D, _B), jnp.float32),
    scratch_types=(
        [pltpu.VMEM_SHARED((_V,), jnp.float32) for _ in range(_NB)]
        + [pltpu.VMEM((_CH,), jnp.float32) for _ in range(_NB)]
        + [
            pltpu.VMEM((_B,), jnp.int32),
            pltpu.SemaphoreType.DMA,
            pltpu.SemaphoreType.DMA,
        ]
    ),
)
def _gather_kernel(wt_hbm, idx_hbm, out_hbm, *refs):
    bufs = refs[:_NB]
    outs = refs[_NB:2 * _NB]
    idx_v, sem_s, sem_g = refs[2 * _NB:]
    c = lax.axis_index("c")
    s = lax.axis_index("s")
    pltpu.sync_copy(idx_hbm, idx_v)
    my_idx = idx_v.at[pl.ds(s * _CH, _CH)]
    k = 0
    while k < _RPC:
        n = min(_NB, _RPC - k)
        stages = []
        for i in range(n):
            @pl.when(s == i)
            def _(k=k, i=i):
                pltpu.async_copy(
                    wt_hbm.at[c * _RPC + k + i], bufs[i], sem_s
                ).wait()
        plsc.subcore_barrier()
        gathers = [
            pltpu.async_copy(bufs[b].at[my_idx], outs[b], sem_g)
            for b in range(n)
        ]
        for b in range(n):
            gathers[b].wait()
            pltpu.sync_copy(
                outs[b], out_hbm.at[c * _RPC + k + b, pl.ds(s * _CH, _CH)]
            )
        plsc.subcore_barrier()
        k += n


def kernel(x, emb_weight):
    out_t = _gather_kernel(emb_weight.T, x.astype(jnp.int32))
    return out_t.T


# speedup vs baseline: 1.0856x; 1.0620x over previous
import functools

import jax
import jax.numpy as jnp
from jax import lax
from jax.experimental import pallas as pl
from jax.experimental.pallas import tpu as pltpu
from jax.experimental.pallas import tpu_sc as plsc

_V = 100000
_D = 64
_B = 16384
_NB = 15
_RPC = 32
_PASSES = (11, 11, 10)

_mesh = plsc.VectorSubcoreMesh(core_axis_name="c", subcore_axis_name="s")


@functools.partial(
    pl.kernel,
    mesh=_mesh,
    compiler_params=pltpu.CompilerParams(use_tc_tiling_on_sc=True),
    out_type=jax.ShapeDtypeStruct((_D, _B), jnp.float32),
    scratch_types=(
        [pltpu.VMEM_SHARED((_V,), jnp.float32) for _ in range(_NB)]
        + [
            pltpu.VMEM((_B,), jnp.int32),
            pltpu.VMEM((_B,), jnp.float32),
            pltpu.SemaphoreType.DMA,
        ]
    ),
)
def _gather_kernel(wt_hbm, idx_hbm, out_hbm, *refs):
    bufs = refs[:_NB]
    idx_v, o_v, sem = refs[_NB:]
    c = lax.axis_index("c")
    s = lax.axis_index("s")
    pltpu.sync_copy(idx_hbm, idx_v)
    k = 0
    for n in _PASSES:
        for i in range(n):
            @pl.when(s == i)
            def _(k=k, i=i):
                d = c * _RPC + k + i
                pltpu.sync_copy(wt_hbm.at[d], bufs[i])
                pltpu.async_copy(bufs[i].at[idx_v], o_v, sem).wait()
                pltpu.sync_copy(o_v, out_hbm.at[d])
        k += n


def kernel(x, emb_weight):
    out_t = _gather_kernel(emb_weight.T, x.astype(jnp.int32))
    return out_t.T
